# baseline (speedup 1.0000x reference)
import functools

import jax
import jax.numpy as jnp
from jax import lax
from jax.experimental import pallas as pl
from jax.experimental.pallas import tpu as pltpu
from jax.experimental.pallas import tpu_sc as plsc

_F32 = jnp.float32
_K = 8
_INF = 3.0e38


def _knn_body(pt_ref, p_ref, out_ref, s_ref):
    pt = pt_ref[0]
    p = p_ref[0]
    br, hw = s_ref.shape
    cp = pt.shape[0]
    d_j = jnp.sum(pt * pt, axis=0, keepdims=True)
    d_i = jnp.sum(p * p, axis=1, keepdims=True)
    r = jnp.dot(p, pt, preferred_element_type=_F32)
    d2 = jnp.maximum((d_i + d_j) - 2.0 * r, 0.0)
    iota_j = lax.broadcasted_iota(jnp.int32, (br, hw), 1)
    key_i = ((lax.bitcast_convert_type(d2, jnp.int32) & jnp.int32(-4096))
             | iota_j) + jnp.int32(1 << 23)
    kv = lax.bitcast_convert_type(key_i, _F32)
    s_ref[...] = kv
    cols = []
    for t in range(_K):
        if t:
            kv = s_ref[...]
        m = jnp.min(kv, axis=1, keepdims=True)
        mi = lax.bitcast_convert_type(m, jnp.int32) - jnp.int32(1 << 23)
        cols.append(mi & jnp.int32(hw - 1))
        if t + 1 < _K:
            s_ref[...] = jnp.where(kv == m, _INF, kv)
    out_ref[0] = jnp.concatenate(cols, axis=1)


def _knn_topk(mats):
    b, cp, hw = mats.shape
    br = min(256, hw)
    p = mats.transpose(0, 2, 1)
    return pl.pallas_call(
        _knn_body,
        grid=(b, hw // br),
        in_specs=[
            pl.BlockSpec((1, cp, hw), lambda n, i: (n, 0, 0)),
            pl.BlockSpec((1, br, cp), lambda n, i: (n, i, 0)),
        ],
        out_specs=pl.BlockSpec((1, br, _K), lambda n, i: (n, i, 0)),
        out_shape=jax.ShapeDtypeStruct((b, hw, _K), jnp.int32),
        scratch_shapes=[pltpu.VMEM((br, hw), _F32)],
        compiler_params=pltpu.CompilerParams(
            dimension_semantics=("parallel", "parallel")),
    )(mats, p)


def _feats_body(h_ref, s_ref, wp_ref, bp_ref, wq_ref, bq_ref, p_ref, q_ref):
    hs = h_ref[0] * s_ref[0]
    dot = functools.partial(jnp.dot, preferred_element_type=_F32)
    p_ref[0] = dot(hs, wp_ref[...]) + bp_ref[...]
    q_ref[0] = dot(hs, wq_ref[...]) + bq_ref[...]


def _feat_tables(h, s_scale, wp, bp, wq, bq):
    n, hw, c = h.shape
    bw = min(512, hw)
    blk = pl.BlockSpec((1, bw, 2 * c), lambda i, j: (i, j, 0))
    return pl.pallas_call(
        _feats_body,
        grid=(n, hw // bw),
        in_specs=[
            pl.BlockSpec((1, bw, c), lambda i, j: (i, j, 0)),
            pl.BlockSpec((1, 1, c), lambda i, j: (i, 0, 0)),
            pl.BlockSpec((c, 2 * c), lambda i, j: (0, 0)),
            pl.BlockSpec((1, 2 * c), lambda i, j: (0, 0)),
            pl.BlockSpec((c, 2 * c), lambda i, j: (0, 0)),
            pl.BlockSpec((1, 2 * c), lambda i, j: (0, 0)),
        ],
        out_specs=[blk, blk],
        out_shape=[jax.ShapeDtypeStruct((n, hw, 2 * c), _F32)] * 2,
        compiler_params=pltpu.CompilerParams(
            dimension_semantics=("parallel", "parallel")),
    )(h, s_scale, wp, bp, wq, bq)


_SC_G = 16


def _sc_body(rows_w, c, p_hbm, q_hbm, gir_hbm, gii_hbm, psr_hbm, psi_hbm,
             idx_r, idx_i, rp, rq, psr_v, psi_v, sem):
    g = _SC_G
    wid = lax.axis_index("s") * 2 + lax.axis_index("c")
    base = wid * rows_w
    nsl = c // 16
    zero = jnp.zeros((16,), _F32)
    init = tuple(zero for _ in range(2 * nsl))

    def chunk(ch, acc):
        nb = base + ch * g
        ib = pl.multiple_of(nb * _K, g * _K)
        pltpu.sync_copy(gir_hbm.at[pl.ds(ib, g * _K)], idx_r)
        pltpu.sync_copy(gii_hbm.at[pl.ds(ib, g * _K)], idx_i)
        h1 = pltpu.async_copy(p_hbm.at[idx_r], rp, sem)
        h2 = pltpu.async_copy(q_hbm.at[idx_i], rq, sem)
        h1.wait()
        h2.wait()

        def node(gg, acc2):
            r0 = gg * _K
            accl = list(acc2)
            for c16 in range(nsl):
                sl = pl.ds(c16 * 16, 16)
                sh = pl.ds(c + c16 * 16, 16)
                ar = rp[r0, sl] - rq[r0, sh]
                ai = rq[r0, sl] - rp[r0, sh]
                for kk in range(1, _K):
                    ar = jnp.maximum(ar, rp[r0 + kk, sl] - rq[r0 + kk, sh])
                    ai = jnp.maximum(ai, rq[r0 + kk, sl] - rp[r0 + kk, sh])
                accl[c16] = accl[c16] + jnp.maximum(ar, 0.0)
                accl[nsl + c16] = accl[nsl + c16] + jnp.maximum(ai, 0.0)
            return tuple(accl)

        return lax.fori_loop(0, g, node, acc)

    acc = lax.fori_loop(0, rows_w // g, chunk, init)
    for c16 in range(nsl):
        sl = pl.ds(c16 * 16, 16)
        psr_v[0, sl] = acc[c16]
        psi_v[0, sl] = acc[nsl + c16]
    pltpu.sync_copy(psr_v, psr_hbm.at[pl.ds(wid, 1)])
    pltpu.sync_copy(psi_v, psi_hbm.at[pl.ds(wid, 1)])


def _sc_gather_max(p, q, gidx_rgb, gidx_ir):
    c2 = p.shape[1]
    c = c2 // 2
    info = plsc.get_sparse_core_info()
    nw = info.num_cores * info.num_subcores
    rows_w = (gidx_rgb.shape[0] // _K) // nw
    g = _SC_G
    kern = pl.kernel(
        functools.partial(_sc_body, rows_w, c),
        out_type=[jax.ShapeDtypeStruct((nw, c), _F32)] * 2,
        mesh=plsc.VectorSubcoreMesh(core_axis_name="c", subcore_axis_name="s"),
        scratch_types=[
            pltpu.VMEM((g * _K,), jnp.int32),
            pltpu.VMEM((g * _K,), jnp.int32),
            pltpu.VMEM((g * _K, c2), _F32),
            pltpu.VMEM((g * _K, c2), _F32),
            pltpu.VMEM((1, c), _F32),
            pltpu.VMEM((1, c), _F32),
            pltpu.SemaphoreType.DMA,
        ],
    )
    return kern(p, q, gidx_rgb, gidx_ir)


def _se_body(hw, psr_ref, psi_ref, s_ref, w1t_ref, b1_ref, w2t_ref, b2_ref,
             out_ref):
    sr = jnp.sum(psr_ref[...], axis=0, keepdims=True)
    si = jnp.sum(psi_ref[...], axis=0, keepdims=True)
    t = jnp.concatenate([sr, si], axis=1) * (1.0 / hw)
    dot = functools.partial(jnp.dot, preferred_element_type=_F32)
    z = jnp.maximum(dot(t, w1t_ref[...]) + b1_ref[...], 0.0)
    u = dot(z, w2t_ref[...]) + b2_ref[...]
    sg = 1.0 / (1.0 + jnp.exp(-u))
    out_ref[0] = s_ref[0] * sg


def _se_update(psr, psi, hw, s_scale, w1t, b1, w2t, b2):
    n = s_scale.shape[0]
    nw, c = psr.shape
    tpn = nw // n
    cm = w1t.shape[1]
    return pl.pallas_call(
        functools.partial(_se_body, hw),
        grid=(n,),
        in_specs=[
            pl.BlockSpec((tpn, c), lambda i: (i, 0)),
            pl.BlockSpec((tpn, c), lambda i: (i, 0)),
            pl.BlockSpec((1, 1, c), lambda i: (i, 0, 0)),
            pl.BlockSpec((2 * c, cm), lambda i: (0, 0)),
            pl.BlockSpec((1, cm), lambda i: (0, 0)),
            pl.BlockSpec((cm, c), lambda i: (0, 0)),
            pl.BlockSpec((1, c), lambda i: (0, 0)),
        ],
        out_specs=pl.BlockSpec((1, 1, c), lambda i: (i, 0, 0)),
        out_shape=jax.ShapeDtypeStruct((n, 1, c), _F32),
        compiler_params=pltpu.CompilerParams(
            dimension_semantics=("parallel",)),
    )(psr, psi, s_scale, w1t, b1, w2t, b2)


def _final_body(x_ref, s_ref, g_ref, out_ref):
    c = s_ref.shape[2]
    scale = 1.0 + g_ref[0, 0] * s_ref[0]
    ri = lax.broadcasted_iota(jnp.int32, (c, c), 0)
    ci = lax.broadcasted_iota(jnp.int32, (c, c), 1)
    diag = jnp.where(ri == ci, jnp.broadcast_to(scale, (c, c)), 0.0)
    y = jnp.dot(diag, x_ref[0], preferred_element_type=_F32,
                precision=lax.Precision.HIGHEST)
    out_ref[0] = jnp.maximum(y, 0.0)


def _final_scale(xf, s_scale, gamma):
    n, c, hw = xf.shape
    return pl.pallas_call(
        _final_body,
        grid=(n,),
        in_specs=[
            pl.BlockSpec((1, c, hw), lambda i: (i, 0, 0)),
            pl.BlockSpec((1, 1, c), lambda i: (i, 0, 0)),
            pl.BlockSpec((1, 1), lambda i: (0, 0)),
        ],
        out_specs=pl.BlockSpec((1, c, hw), lambda i: (i, 0, 0)),
        out_shape=jax.ShapeDtypeStruct((n, c, hw), _F32),
        compiler_params=pltpu.CompilerParams(
            dimension_semantics=("parallel",)),
    )(xf, s_scale, gamma)


def kernel(cnn_encoder_output, rgb, ir, gnn_iterations, k,
           rgb_g_W, rgb_g_b, ir_g_W, ir_g_b,
           se_W1, se_b1, se_W2, se_b2, gamma):
    x = cnn_encoder_output
    n, c, h_dim, w_dim = x.shape
    hw = h_dim * w_dim
    xf = x.reshape(n, c, hw)

    rgb_t = rgb.reshape(n, rgb.shape[1], hw)
    ir_t = ir.reshape(n, ir.shape[1], hw)
    ir_pad = jnp.concatenate(
        [ir_t, jnp.zeros((n, rgb.shape[1] - ir.shape[1], hw), _F32)], axis=1)
    gidx = []
    for b in range(n):
        mats_b = jnp.concatenate(
            [rgb_t[b:b + 1], ir_pad[b:b + 1]], axis=0)
        idx_b = _knn_topk(mats_b) + jnp.int32(b * hw)
        gidx.append((idx_b[0].reshape(hw * _K), idx_b[1].reshape(hw * _K)))

    h0 = xf.transpose(0, 2, 1)
    wt_rgb = rgb_g_W.T
    wt_ir = ir_g_W.T
    zc = jnp.zeros((1, c), _F32)
    wp = jnp.concatenate([wt_rgb[:c] + wt_rgb[c:], wt_ir[c:]], axis=1)
    bp = jnp.concatenate([rgb_g_b.reshape(1, c), zc], axis=1)
    wq = jnp.concatenate([wt_ir[:c] + wt_ir[c:], wt_rgb[c:]], axis=1)
    bq = jnp.concatenate([ir_g_b.reshape(1, c), zc], axis=1)
    w1t = se_W1.T
    b1 = se_b1.reshape(1, -1)
    w2t = se_W2.T
    b2 = se_b2.reshape(1, c)

    def body(_, s_scale):
        p, q = _feat_tables(h0, s_scale, wp, bp, wq, bq)
        pf = p.reshape(n * hw, 2 * c)
        qf = q.reshape(n * hw, 2 * c)
        parts = [_sc_gather_max(pf, qf, gr, gi) for gr, gi in gidx]
        psr = jnp.concatenate([pr for pr, _ in parts], axis=0)
        psi = jnp.concatenate([pi for _, pi in parts], axis=0)
        return _se_update(psr, psi, hw, s_scale, w1t, b1, w2t, b2)

    s_scale = lax.fori_loop(0, gnn_iterations, body,
                            jnp.ones((n, 1, c), _F32))

    out = _final_scale(xf, s_scale, gamma.reshape(1, 1).astype(_F32))
    return out.reshape(n, c, h_dim, w_dim)


# speedup vs baseline: 1.0519x; 1.0519x over previous
import functools

import jax
import jax.numpy as jnp
from jax import lax
from jax.experimental import pallas as pl
from jax.experimental.pallas import tpu as pltpu
from jax.experimental.pallas import tpu_sc as plsc

_F32 = jnp.float32
_K = 8
_INF = 3.0e38


def _knn_body(pt_ref, p_ref, out_ref, s_ref):
    pt = pt_ref[0]
    p = p_ref[0]
    br, hw = s_ref.shape
    cp = pt.shape[0]
    d_j = jnp.sum(pt * pt, axis=0, keepdims=True)
    d_i = jnp.sum(p * p, axis=1, keepdims=True)
    r = jnp.dot(p, pt, preferred_element_type=_F32)
    d2 = jnp.maximum((d_i + d_j) - 2.0 * r, 0.0)
    iota_j = lax.broadcasted_iota(jnp.int32, (br, hw), 1)
    key_i = ((lax.bitcast_convert_type(d2, jnp.int32) & jnp.int32(-4096))
             | iota_j) + jnp.int32(1 << 23)
    kv = lax.bitcast_convert_type(key_i, _F32)
    s_ref[...] = kv
    cols = []
    for t in range(_K):
        if t:
            kv = s_ref[...]
        m = jnp.min(kv, axis=1, keepdims=True)
        mi = lax.bitcast_convert_type(m, jnp.int32) - jnp.int32(1 << 23)
        cols.append(mi & jnp.int32(hw - 1))
        if t + 1 < _K:
            s_ref[...] = jnp.where(kv == m, _INF, kv)
    out_ref[0] = jnp.concatenate(cols, axis=1)


def _knn_topk(mats):
    b, cp, hw = mats.shape
    br = min(256, hw)
    p = mats.transpose(0, 2, 1)
    return pl.pallas_call(
        _knn_body,
        grid=(b, hw // br),
        in_specs=[
            pl.BlockSpec((1, cp, hw), lambda n, i: (n, 0, 0)),
            pl.BlockSpec((1, br, cp), lambda n, i: (n, i, 0)),
        ],
        out_specs=pl.BlockSpec((1, br, _K), lambda n, i: (n, i, 0)),
        out_shape=jax.ShapeDtypeStruct((b, hw, _K), jnp.int32),
        scratch_shapes=[pltpu.VMEM((br, hw), _F32)],
        compiler_params=pltpu.CompilerParams(
            dimension_semantics=("parallel", "parallel")),
    )(mats, p)


def _feats_body(h_ref, s_ref, wp_ref, bp_ref, wq_ref, bq_ref, p_ref, q_ref):
    hs = h_ref[0] * s_ref[0]
    dot = functools.partial(jnp.dot, preferred_element_type=_F32)
    p_ref[0] = dot(hs, wp_ref[...]) + bp_ref[...]
    q_ref[0] = dot(hs, wq_ref[...]) + bq_ref[...]


def _feat_tables(h, s_scale, wp, bp, wq, bq):
    n, hw, c = h.shape
    bw = min(512, hw)
    blk = pl.BlockSpec((1, bw, 2 * c), lambda i, j: (i, j, 0))
    return pl.pallas_call(
        _feats_body,
        grid=(n, hw // bw),
        in_specs=[
            pl.BlockSpec((1, bw, c), lambda i, j: (i, j, 0)),
            pl.BlockSpec((1, 1, c), lambda i, j: (i, 0, 0)),
            pl.BlockSpec((c, 2 * c), lambda i, j: (0, 0)),
            pl.BlockSpec((1, 2 * c), lambda i, j: (0, 0)),
            pl.BlockSpec((c, 2 * c), lambda i, j: (0, 0)),
            pl.BlockSpec((1, 2 * c), lambda i, j: (0, 0)),
        ],
        out_specs=[blk, blk],
        out_shape=[jax.ShapeDtypeStruct((n, hw, 2 * c), _F32)] * 2,
        compiler_params=pltpu.CompilerParams(
            dimension_semantics=("parallel", "parallel")),
    )(h, s_scale, wp, bp, wq, bq)


_SC_G = 16


def _sc_body(rows_w, c, p_hbm, q_hbm, gir_hbm, gii_hbm, psr_hbm, psi_hbm,
             idx_r, idx_i, rp, rq, psr_v, psi_v, sem):
    g = _SC_G
    wid = lax.axis_index("s") * 2 + lax.axis_index("c")
    base = wid * rows_w
    nsl = c // 16
    zero = jnp.zeros((16,), _F32)
    init = tuple(zero for _ in range(2 * nsl))

    def chunk(ch, acc):
        nb = base + ch * g
        ib = pl.multiple_of(nb * _K, g * _K)
        pltpu.sync_copy(gir_hbm.at[pl.ds(ib, g * _K)], idx_r)
        pltpu.sync_copy(gii_hbm.at[pl.ds(ib, g * _K)], idx_i)
        h1 = pltpu.async_copy(p_hbm.at[idx_r], rp, sem)
        h2 = pltpu.async_copy(q_hbm.at[idx_i], rq, sem)
        h1.wait()
        h2.wait()

        def node(gg, acc2):
            r0 = gg * _K
            accl = list(acc2)
            for c16 in range(nsl):
                sl = pl.ds(c16 * 16, 16)
                sh = pl.ds(c + c16 * 16, 16)
                ar = rp[r0, sl] - rq[r0, sh]
                ai = rq[r0, sl] - rp[r0, sh]
                for kk in range(1, _K):
                    ar = jnp.maximum(ar, rp[r0 + kk, sl] - rq[r0 + kk, sh])
                    ai = jnp.maximum(ai, rq[r0 + kk, sl] - rp[r0 + kk, sh])
                accl[c16] = accl[c16] + jnp.maximum(ar, 0.0)
                accl[nsl + c16] = accl[nsl + c16] + jnp.maximum(ai, 0.0)
            return tuple(accl)

        return lax.fori_loop(0, g, node, acc)

    acc = lax.fori_loop(0, rows_w // g, chunk, init)
    for c16 in range(nsl):
        sl = pl.ds(c16 * 16, 16)
        psr_v[0, sl] = acc[c16]
        psi_v[0, sl] = acc[nsl + c16]
    pltpu.sync_copy(psr_v, psr_hbm.at[pl.ds(wid, 1)])
    pltpu.sync_copy(psi_v, psi_hbm.at[pl.ds(wid, 1)])


def _sc_gather_max(p, q, gidx_rgb, gidx_ir):
    c2 = p.shape[1]
    c = c2 // 2
    info = plsc.get_sparse_core_info()
    nw = info.num_cores * info.num_subcores
    rows_w = (gidx_rgb.shape[0] // _K) // nw
    g = _SC_G
    kern = pl.kernel(
        functools.partial(_sc_body, rows_w, c),
        out_type=[jax.ShapeDtypeStruct((nw, c), _F32)] * 2,
        mesh=plsc.VectorSubcoreMesh(core_axis_name="c", subcore_axis_name="s"),
        scratch_types=[
            pltpu.VMEM((g * _K,), jnp.int32),
            pltpu.VMEM((g * _K,), jnp.int32),
            pltpu.VMEM((g * _K, c2), _F32),
            pltpu.VMEM((g * _K, c2), _F32),
            pltpu.VMEM((1, c), _F32),
            pltpu.VMEM((1, c), _F32),
            pltpu.SemaphoreType.DMA,
        ],
    )
    return kern(p, q, gidx_rgb, gidx_ir)


def _se_body(hw, psr_ref, psi_ref, s_ref, w1t_ref, b1_ref, w2t_ref, b2_ref,
             out_ref):
    sr = jnp.sum(psr_ref[...], axis=0, keepdims=True)
    si = jnp.sum(psi_ref[...], axis=0, keepdims=True)
    t = jnp.concatenate([sr, si], axis=1) * (1.0 / hw)
    dot = functools.partial(jnp.dot, preferred_element_type=_F32)
    z = jnp.maximum(dot(t, w1t_ref[...]) + b1_ref[...], 0.0)
    u = dot(z, w2t_ref[...]) + b2_ref[...]
    sg = 1.0 / (1.0 + jnp.exp(-u))
    out_ref[0] = s_ref[0] * sg


def _se_update(psr, psi, hw, s_scale, w1t, b1, w2t, b2):
    n = s_scale.shape[0]
    nw, c = psr.shape
    tpn = nw // n
    cm = w1t.shape[1]
    return pl.pallas_call(
        functools.partial(_se_body, hw),
        grid=(n,),
        in_specs=[
            pl.BlockSpec((tpn, c), lambda i: (i, 0)),
            pl.BlockSpec((tpn, c), lambda i: (i, 0)),
            pl.BlockSpec((1, 1, c), lambda i: (i, 0, 0)),
            pl.BlockSpec((2 * c, cm), lambda i: (0, 0)),
            pl.BlockSpec((1, cm), lambda i: (0, 0)),
            pl.BlockSpec((cm, c), lambda i: (0, 0)),
            pl.BlockSpec((1, c), lambda i: (0, 0)),
        ],
        out_specs=pl.BlockSpec((1, 1, c), lambda i: (i, 0, 0)),
        out_shape=jax.ShapeDtypeStruct((n, 1, c), _F32),
        compiler_params=pltpu.CompilerParams(
            dimension_semantics=("parallel",)),
    )(psr, psi, s_scale, w1t, b1, w2t, b2)


def _final_body(x_ref, s_ref, g_ref, out_ref):
    c = s_ref.shape[2]
    scale = 1.0 + g_ref[0, 0] * s_ref[0]
    ri = lax.broadcasted_iota(jnp.int32, (c, c), 0)
    ci = lax.broadcasted_iota(jnp.int32, (c, c), 1)
    diag = jnp.where(ri == ci, jnp.broadcast_to(scale, (c, c)), 0.0)
    y = jnp.dot(diag, x_ref[0], preferred_element_type=_F32,
                precision=lax.Precision.HIGHEST)
    out_ref[0] = jnp.maximum(y, 0.0)


def _final_scale(xf, s_scale, gamma):
    n, c, hw = xf.shape
    return pl.pallas_call(
        _final_body,
        grid=(n,),
        in_specs=[
            pl.BlockSpec((1, c, hw), lambda i: (i, 0, 0)),
            pl.BlockSpec((1, 1, c), lambda i: (i, 0, 0)),
            pl.BlockSpec((1, 1), lambda i: (0, 0)),
        ],
        out_specs=pl.BlockSpec((1, c, hw), lambda i: (i, 0, 0)),
        out_shape=jax.ShapeDtypeStruct((n, c, hw), _F32),
        compiler_params=pltpu.CompilerParams(
            dimension_semantics=("parallel",)),
    )(xf, s_scale, gamma)


def kernel(cnn_encoder_output, rgb, ir, gnn_iterations, k,
           rgb_g_W, rgb_g_b, ir_g_W, ir_g_b,
           se_W1, se_b1, se_W2, se_b2, gamma):
    x = cnn_encoder_output
    n, c, h_dim, w_dim = x.shape
    hw = h_dim * w_dim
    xf = x.reshape(n, c, hw)

    rgb_t = rgb.reshape(n, rgb.shape[1], hw)
    ir_t = ir.reshape(n, ir.shape[1], hw)
    ir_pad = jnp.concatenate(
        [ir_t, jnp.zeros((n, rgb.shape[1] - ir.shape[1], hw), _F32)], axis=1)
    mats = jnp.concatenate([rgb_t, ir_pad], axis=0)
    idx_all = _knn_topk(mats)
    idx_rgb, idx_ir = idx_all[:n], idx_all[n:]

    offs = (jnp.arange(n, dtype=jnp.int32) * hw)[:, None, None]
    gidx_rgb = (idx_rgb + offs).reshape(n * hw * _K)
    gidx_ir = (idx_ir + offs).reshape(n * hw * _K)

    h0 = xf.transpose(0, 2, 1)
    wt_rgb = rgb_g_W.T
    wt_ir = ir_g_W.T
    zc = jnp.zeros((1, c), _F32)
    wp = jnp.concatenate([wt_rgb[:c] + wt_rgb[c:], wt_ir[c:]], axis=1)
    bp = jnp.concatenate([rgb_g_b.reshape(1, c), zc], axis=1)
    wq = jnp.concatenate([wt_ir[:c] + wt_ir[c:], wt_rgb[c:]], axis=1)
    bq = jnp.concatenate([ir_g_b.reshape(1, c), zc], axis=1)
    w1t = se_W1.T
    b1 = se_b1.reshape(1, -1)
    w2t = se_W2.T
    b2 = se_b2.reshape(1, c)

    def body(_, s_scale):
        p, q = _feat_tables(h0, s_scale, wp, bp, wq, bq)
        psr, psi = _sc_gather_max(
            p.reshape(n * hw, 2 * c), q.reshape(n * hw, 2 * c),
            gidx_rgb, gidx_ir)
        return _se_update(psr, psi, hw, s_scale, w1t, b1, w2t, b2)

    s_scale = lax.fori_loop(0, gnn_iterations, body,
                            jnp.ones((n, 1, c), _F32))

    out = _final_scale(xf, s_scale, gamma.reshape(1, 1).astype(_F32))
    return out.reshape(n, c, h_dim, w_dim)


# speedup vs baseline: 1.0735x; 1.0206x over previous
import functools

import jax
import jax.numpy as jnp
from jax import lax
from jax.experimental import pallas as pl
from jax.experimental.pallas import tpu as pltpu
from jax.experimental.pallas import tpu_sc as plsc

_F32 = jnp.float32
_K = 8
_INF = 3.0e38


def _knn_body(pt_ref, p_ref, out_ref, s_ref):
    pt = pt_ref[0]
    p = p_ref[0]
    br, hw = s_ref.shape
    cp = pt.shape[0]
    d_j = jnp.sum(pt * pt, axis=0, keepdims=True)
    d_i = jnp.sum(p * p, axis=1, keepdims=True)
    r = jnp.dot(p, pt, preferred_element_type=_F32)
    d2 = jnp.maximum((d_i + d_j) - 2.0 * r, 0.0)
    iota_j = lax.broadcasted_iota(jnp.int32, (br, hw), 1)
    key_i = ((lax.bitcast_convert_type(d2, jnp.int32) & jnp.int32(-4096))
             | iota_j) + jnp.int32(1 << 23)
    kv = lax.bitcast_convert_type(key_i, _F32)
    s_ref[...] = kv
    cols = []
    for t in range(_K):
        if t:
            kv = s_ref[...]
        m = jnp.min(kv, axis=1, keepdims=True)
        mi = lax.bitcast_convert_type(m, jnp.int32) - jnp.int32(1 << 23)
        cols.append(mi & jnp.int32(hw - 1))
        if t + 1 < _K:
            s_ref[...] = jnp.where(kv == m, _INF, kv)
    out_ref[0] = jnp.concatenate(cols, axis=1)


def _knn_topk(mats):
    b, cp, hw = mats.shape
    br = min(512, hw)
    p = mats.transpose(0, 2, 1)
    return pl.pallas_call(
        _knn_body,
        grid=(b, hw // br),
        in_specs=[
            pl.BlockSpec((1, cp, hw), lambda n, i: (n, 0, 0)),
            pl.BlockSpec((1, br, cp), lambda n, i: (n, i, 0)),
        ],
        out_specs=pl.BlockSpec((1, br, _K), lambda n, i: (n, i, 0)),
        out_shape=jax.ShapeDtypeStruct((b, hw, _K), jnp.int32),
        scratch_shapes=[pltpu.VMEM((br, hw), _F32)],
        compiler_params=pltpu.CompilerParams(
            dimension_semantics=("parallel", "parallel")),
    )(mats, p)


def _feats_body(h_ref, s_ref, wp_ref, bp_ref, wq_ref, bq_ref, p_ref, q_ref):
    hs = h_ref[0] * s_ref[0]
    dot = functools.partial(jnp.dot, preferred_element_type=_F32)
    p_ref[0] = dot(hs, wp_ref[...]) + bp_ref[...]
    q_ref[0] = dot(hs, wq_ref[...]) + bq_ref[...]


def _feat_tables(h, s_scale, wp, bp, wq, bq):
    n, hw, c = h.shape
    bw = min(512, hw)
    blk = pl.BlockSpec((1, bw, 2 * c), lambda i, j: (i, j, 0))
    return pl.pallas_call(
        _feats_body,
        grid=(n, hw // bw),
        in_specs=[
            pl.BlockSpec((1, bw, c), lambda i, j: (i, j, 0)),
            pl.BlockSpec((1, 1, c), lambda i, j: (i, 0, 0)),
            pl.BlockSpec((c, 2 * c), lambda i, j: (0, 0)),
            pl.BlockSpec((1, 2 * c), lambda i, j: (0, 0)),
            pl.BlockSpec((c, 2 * c), lambda i, j: (0, 0)),
            pl.BlockSpec((1, 2 * c), lambda i, j: (0, 0)),
        ],
        out_specs=[blk, blk],
        out_shape=[jax.ShapeDtypeStruct((n, hw, 2 * c), _F32)] * 2,
        compiler_params=pltpu.CompilerParams(
            dimension_semantics=("parallel", "parallel")),
    )(h, s_scale, wp, bp, wq, bq)


_SC_G = 16


def _sc_body(rows_w, c, p_hbm, q_hbm, gir_hbm, gii_hbm, psr_hbm, psi_hbm,
             idx_r, idx_i, rp, rq, psr_v, psi_v, sem):
    g = _SC_G
    wid = lax.axis_index("s") * 2 + lax.axis_index("c")
    base = wid * rows_w
    nsl = c // 16
    zero = jnp.zeros((16,), _F32)
    init = tuple(zero for _ in range(2 * nsl))

    def chunk(ch, acc):
        nb = base + ch * g
        ib = pl.multiple_of(nb * _K, g * _K)
        pltpu.sync_copy(gir_hbm.at[pl.ds(ib, g * _K)], idx_r)
        pltpu.sync_copy(gii_hbm.at[pl.ds(ib, g * _K)], idx_i)
        h1 = pltpu.async_copy(p_hbm.at[idx_r], rp, sem)
        h2 = pltpu.async_copy(q_hbm.at[idx_i], rq, sem)
        h1.wait()
        h2.wait()

        def node(gg, acc2):
            r0 = gg * _K
            accl = list(acc2)
            for c16 in range(nsl):
                sl = pl.ds(c16 * 16, 16)
                sh = pl.ds(c + c16 * 16, 16)
                ar = rp[r0, sl] - rq[r0, sh]
                ai = rq[r0, sl] - rp[r0, sh]
                for kk in range(1, _K):
                    ar = jnp.maximum(ar, rp[r0 + kk, sl] - rq[r0 + kk, sh])
                    ai = jnp.maximum(ai, rq[r0 + kk, sl] - rp[r0 + kk, sh])
                accl[c16] = accl[c16] + jnp.maximum(ar, 0.0)
                accl[nsl + c16] = accl[nsl + c16] + jnp.maximum(ai, 0.0)
            return tuple(accl)

        return lax.fori_loop(0, g, node, acc)

    acc = lax.fori_loop(0, rows_w // g, chunk, init)
    for c16 in range(nsl):
        sl = pl.ds(c16 * 16, 16)
        psr_v[0, sl] = acc[c16]
        psi_v[0, sl] = acc[nsl + c16]
    pltpu.sync_copy(psr_v, psr_hbm.at[pl.ds(wid, 1)])
    pltpu.sync_copy(psi_v, psi_hbm.at[pl.ds(wid, 1)])


def _sc_gather_max(p, q, gidx_rgb, gidx_ir):
    c2 = p.shape[1]
    c = c2 // 2
    info = plsc.get_sparse_core_info()
    nw = info.num_cores * info.num_subcores
    rows_w = (gidx_rgb.shape[0] // _K) // nw
    g = _SC_G
    kern = pl.kernel(
        functools.partial(_sc_body, rows_w, c),
        out_type=[jax.ShapeDtypeStruct((nw, c), _F32)] * 2,
        mesh=plsc.VectorSubcoreMesh(core_axis_name="c", subcore_axis_name="s"),
        scratch_types=[
            pltpu.VMEM((g * _K,), jnp.int32),
            pltpu.VMEM((g * _K,), jnp.int32),
            pltpu.VMEM((g * _K, c2), _F32),
            pltpu.VMEM((g * _K, c2), _F32),
            pltpu.VMEM((1, c), _F32),
            pltpu.VMEM((1, c), _F32),
            pltpu.SemaphoreType.DMA,
        ],
    )
    return kern(p, q, gidx_rgb, gidx_ir)


def _se_body(hw, psr_ref, psi_ref, s_ref, w1t_ref, b1_ref, w2t_ref, b2_ref,
             out_ref):
    sr = jnp.sum(psr_ref[...], axis=0, keepdims=True)
    si = jnp.sum(psi_ref[...], axis=0, keepdims=True)
    t = jnp.concatenate([sr, si], axis=1) * (1.0 / hw)
    dot = functools.partial(jnp.dot, preferred_element_type=_F32)
    z = jnp.maximum(dot(t, w1t_ref[...]) + b1_ref[...], 0.0)
    u = dot(z, w2t_ref[...]) + b2_ref[...]
    sg = 1.0 / (1.0 + jnp.exp(-u))
    out_ref[0] = s_ref[0] * sg


def _se_update(psr, psi, hw, s_scale, w1t, b1, w2t, b2):
    n = s_scale.shape[0]
    nw, c = psr.shape
    tpn = nw // n
    cm = w1t.shape[1]
    return pl.pallas_call(
        functools.partial(_se_body, hw),
        grid=(n,),
        in_specs=[
            pl.BlockSpec((tpn, c), lambda i: (i, 0)),
            pl.BlockSpec((tpn, c), lambda i: (i, 0)),
            pl.BlockSpec((1, 1, c), lambda i: (i, 0, 0)),
            pl.BlockSpec((2 * c, cm), lambda i: (0, 0)),
            pl.BlockSpec((1, cm), lambda i: (0, 0)),
            pl.BlockSpec((cm, c), lambda i: (0, 0)),
            pl.BlockSpec((1, c), lambda i: (0, 0)),
        ],
        out_specs=pl.BlockSpec((1, 1, c), lambda i: (i, 0, 0)),
        out_shape=jax.ShapeDtypeStruct((n, 1, c), _F32),
        compiler_params=pltpu.CompilerParams(
            dimension_semantics=("parallel",)),
    )(psr, psi, s_scale, w1t, b1, w2t, b2)


def _final_body(x_ref, s_ref, g_ref, out_ref):
    c = s_ref.shape[2]
    scale = 1.0 + g_ref[0, 0] * s_ref[0]
    ri = lax.broadcasted_iota(jnp.int32, (c, c), 0)
    ci = lax.broadcasted_iota(jnp.int32, (c, c), 1)
    diag = jnp.where(ri == ci, jnp.broadcast_to(scale, (c, c)), 0.0)
    y = jnp.dot(diag, x_ref[0], preferred_element_type=_F32,
                precision=lax.Precision.HIGHEST)
    out_ref[0] = jnp.maximum(y, 0.0)


def _final_scale(xf, s_scale, gamma):
    n, c, hw = xf.shape
    return pl.pallas_call(
        _final_body,
        grid=(n,),
        in_specs=[
            pl.BlockSpec((1, c, hw), lambda i: (i, 0, 0)),
            pl.BlockSpec((1, 1, c), lambda i: (i, 0, 0)),
            pl.BlockSpec((1, 1), lambda i: (0, 0)),
        ],
        out_specs=pl.BlockSpec((1, c, hw), lambda i: (i, 0, 0)),
        out_shape=jax.ShapeDtypeStruct((n, c, hw), _F32),
        compiler_params=pltpu.CompilerParams(
            dimension_semantics=("parallel",)),
    )(xf, s_scale, gamma)


def kernel(cnn_encoder_output, rgb, ir, gnn_iterations, k,
           rgb_g_W, rgb_g_b, ir_g_W, ir_g_b,
           se_W1, se_b1, se_W2, se_b2, gamma):
    x = cnn_encoder_output
    n, c, h_dim, w_dim = x.shape
    hw = h_dim * w_dim
    xf = x.reshape(n, c, hw)

    rgb_t = rgb.reshape(n, rgb.shape[1], hw)
    ir_t = ir.reshape(n, ir.shape[1], hw)
    ir_pad = jnp.concatenate(
        [ir_t, jnp.zeros((n, rgb.shape[1] - ir.shape[1], hw), _F32)], axis=1)
    mats = jnp.concatenate([rgb_t, ir_pad], axis=0)
    idx_all = _knn_topk(mats)
    idx_rgb, idx_ir = idx_all[:n], idx_all[n:]

    offs = (jnp.arange(n, dtype=jnp.int32) * hw)[:, None, None]
    gidx_rgb = (idx_rgb + offs).reshape(n * hw * _K)
    gidx_ir = (idx_ir + offs).reshape(n * hw * _K)

    h0 = xf.transpose(0, 2, 1)
    wt_rgb = rgb_g_W.T
    wt_ir = ir_g_W.T
    zc = jnp.zeros((1, c), _F32)
    wp = jnp.concatenate([wt_rgb[:c] + wt_rgb[c:], wt_ir[c:]], axis=1)
    bp = jnp.concatenate([rgb_g_b.reshape(1, c), zc], axis=1)
    wq = jnp.concatenate([wt_ir[:c] + wt_ir[c:], wt_rgb[c:]], axis=1)
    bq = jnp.concatenate([ir_g_b.reshape(1, c), zc], axis=1)
    w1t = se_W1.T
    b1 = se_b1.reshape(1, -1)
    w2t = se_W2.T
    b2 = se_b2.reshape(1, c)

    def body(_, s_scale):
        p, q = _feat_tables(h0, s_scale, wp, bp, wq, bq)
        psr, psi = _sc_gather_max(
            p.reshape(n * hw, 2 * c), q.reshape(n * hw, 2 * c),
            gidx_rgb, gidx_ir)
        return _se_update(psr, psi, hw, s_scale, w1t, b1, w2t, b2)

    s_scale = lax.fori_loop(0, gnn_iterations, body,
                            jnp.ones((n, 1, c), _F32))

    out = _final_scale(xf, s_scale, gamma.reshape(1, 1).astype(_F32))
    return out.reshape(n, c, h_dim, w_dim)


# speedup vs baseline: 1.0890x; 1.0144x over previous
import functools

import jax
import jax.numpy as jnp
from jax import lax
from jax.experimental import pallas as pl
from jax.experimental.pallas import tpu as pltpu
from jax.experimental.pallas import tpu_sc as plsc

_F32 = jnp.float32
_K = 8
_INF = 3.0e38


def _knn_body(pt_ref, p_ref, out_ref, s_ref):
    pt = pt_ref[0]
    p = p_ref[0]
    br, hw = s_ref.shape
    cp = pt.shape[0]
    d_j = jnp.sum(pt * pt, axis=0, keepdims=True)
    d_i = jnp.sum(p * p, axis=1, keepdims=True)
    r = jnp.dot(p, pt, preferred_element_type=_F32)
    d2 = jnp.maximum((d_i + d_j) - 2.0 * r, 0.0)
    iota_j = lax.broadcasted_iota(jnp.int32, (br, hw), 1)
    key_i = ((lax.bitcast_convert_type(d2, jnp.int32) & jnp.int32(-4096))
             | iota_j) + jnp.int32(1 << 23)
    kv = lax.bitcast_convert_type(key_i, _F32)
    s_ref[...] = kv
    cols = []
    for t in range(_K):
        if t:
            kv = s_ref[...]
        m = jnp.min(kv, axis=1, keepdims=True)
        mi = lax.bitcast_convert_type(m, jnp.int32) - jnp.int32(1 << 23)
        cols.append(mi & jnp.int32(hw - 1))
        if t + 1 < _K:
            s_ref[...] = jnp.where(kv == m, _INF, kv)
    out_ref[0] = jnp.concatenate(cols, axis=1)


def _knn_topk(mats):
    b, cp, hw = mats.shape
    br = min(1024, hw)
    p = mats.transpose(0, 2, 1)
    return pl.pallas_call(
        _knn_body,
        grid=(b, hw // br),
        in_specs=[
            pl.BlockSpec((1, cp, hw), lambda n, i: (n, 0, 0)),
            pl.BlockSpec((1, br, cp), lambda n, i: (n, i, 0)),
        ],
        out_specs=pl.BlockSpec((1, br, _K), lambda n, i: (n, i, 0)),
        out_shape=jax.ShapeDtypeStruct((b, hw, _K), jnp.int32),
        scratch_shapes=[pltpu.VMEM((br, hw), _F32)],
        compiler_params=pltpu.CompilerParams(
            dimension_semantics=("parallel", "parallel")),
    )(mats, p)


def _feats_body(h_ref, s_ref, wp_ref, bp_ref, wq_ref, bq_ref, p_ref, q_ref):
    hs = h_ref[0] * s_ref[0]
    dot = functools.partial(jnp.dot, preferred_element_type=_F32)
    p_ref[0] = dot(hs, wp_ref[...]) + bp_ref[...]
    q_ref[0] = dot(hs, wq_ref[...]) + bq_ref[...]


def _feat_tables(h, s_scale, wp, bp, wq, bq):
    n, hw, c = h.shape
    bw = min(512, hw)
    blk = pl.BlockSpec((1, bw, 2 * c), lambda i, j: (i, j, 0))
    return pl.pallas_call(
        _feats_body,
        grid=(n, hw // bw),
        in_specs=[
            pl.BlockSpec((1, bw, c), lambda i, j: (i, j, 0)),
            pl.BlockSpec((1, 1, c), lambda i, j: (i, 0, 0)),
            pl.BlockSpec((c, 2 * c), lambda i, j: (0, 0)),
            pl.BlockSpec((1, 2 * c), lambda i, j: (0, 0)),
            pl.BlockSpec((c, 2 * c), lambda i, j: (0, 0)),
            pl.BlockSpec((1, 2 * c), lambda i, j: (0, 0)),
        ],
        out_specs=[blk, blk],
        out_shape=[jax.ShapeDtypeStruct((n, hw, 2 * c), _F32)] * 2,
        compiler_params=pltpu.CompilerParams(
            dimension_semantics=("parallel", "parallel")),
    )(h, s_scale, wp, bp, wq, bq)


_SC_G = 16


def _sc_body(rows_w, c, p_hbm, q_hbm, gir_hbm, gii_hbm, psr_hbm, psi_hbm,
             idx_r, idx_i, rp, rq, psr_v, psi_v, sem):
    g = _SC_G
    wid = lax.axis_index("s") * 2 + lax.axis_index("c")
    base = wid * rows_w
    nsl = c // 16
    zero = jnp.zeros((16,), _F32)
    init = tuple(zero for _ in range(2 * nsl))

    def chunk(ch, acc):
        nb = base + ch * g
        ib = pl.multiple_of(nb * _K, g * _K)
        pltpu.sync_copy(gir_hbm.at[pl.ds(ib, g * _K)], idx_r)
        pltpu.sync_copy(gii_hbm.at[pl.ds(ib, g * _K)], idx_i)
        h1 = pltpu.async_copy(p_hbm.at[idx_r], rp, sem)
        h2 = pltpu.async_copy(q_hbm.at[idx_i], rq, sem)
        h1.wait()
        h2.wait()

        def node(gg, acc2):
            r0 = gg * _K
            accl = list(acc2)
            for c16 in range(nsl):
                sl = pl.ds(c16 * 16, 16)
                sh = pl.ds(c + c16 * 16, 16)
                ar = rp[r0, sl] - rq[r0, sh]
                ai = rq[r0, sl] - rp[r0, sh]
                for kk in range(1, _K):
                    ar = jnp.maximum(ar, rp[r0 + kk, sl] - rq[r0 + kk, sh])
                    ai = jnp.maximum(ai, rq[r0 + kk, sl] - rp[r0 + kk, sh])
                accl[c16] = accl[c16] + jnp.maximum(ar, 0.0)
                accl[nsl + c16] = accl[nsl + c16] + jnp.maximum(ai, 0.0)
            return tuple(accl)

        return lax.fori_loop(0, g, node, acc)

    acc = lax.fori_loop(0, rows_w // g, chunk, init)
    for c16 in range(nsl):
        sl = pl.ds(c16 * 16, 16)
        psr_v[0, sl] = acc[c16]
        psi_v[0, sl] = acc[nsl + c16]
    pltpu.sync_copy(psr_v, psr_hbm.at[pl.ds(wid, 1)])
    pltpu.sync_copy(psi_v, psi_hbm.at[pl.ds(wid, 1)])


def _sc_gather_max(p, q, gidx_rgb, gidx_ir):
    c2 = p.shape[1]
    c = c2 // 2
    info = plsc.get_sparse_core_info()
    nw = info.num_cores * info.num_subcores
    rows_w = (gidx_rgb.shape[0] // _K) // nw
    g = _SC_G
    kern = pl.kernel(
        functools.partial(_sc_body, rows_w, c),
        out_type=[jax.ShapeDtypeStruct((nw, c), _F32)] * 2,
        mesh=plsc.VectorSubcoreMesh(core_axis_name="c", subcore_axis_name="s"),
        scratch_types=[
            pltpu.VMEM((g * _K,), jnp.int32),
            pltpu.VMEM((g * _K,), jnp.int32),
            pltpu.VMEM((g * _K, c2), _F32),
            pltpu.VMEM((g * _K, c2), _F32),
            pltpu.VMEM((1, c), _F32),
            pltpu.VMEM((1, c), _F32),
            pltpu.SemaphoreType.DMA,
        ],
    )
    return kern(p, q, gidx_rgb, gidx_ir)


def _se_body(hw, psr_ref, psi_ref, s_ref, w1t_ref, b1_ref, w2t_ref, b2_ref,
             out_ref):
    sr = jnp.sum(psr_ref[...], axis=0, keepdims=True)
    si = jnp.sum(psi_ref[...], axis=0, keepdims=True)
    t = jnp.concatenate([sr, si], axis=1) * (1.0 / hw)
    dot = functools.partial(jnp.dot, preferred_element_type=_F32)
    z = jnp.maximum(dot(t, w1t_ref[...]) + b1_ref[...], 0.0)
    u = dot(z, w2t_ref[...]) + b2_ref[...]
    sg = 1.0 / (1.0 + jnp.exp(-u))
    out_ref[0] = s_ref[0] * sg


def _se_update(psr, psi, hw, s_scale, w1t, b1, w2t, b2):
    n = s_scale.shape[0]
    nw, c = psr.shape
    tpn = nw // n
    cm = w1t.shape[1]
    return pl.pallas_call(
        functools.partial(_se_body, hw),
        grid=(n,),
        in_specs=[
            pl.BlockSpec((tpn, c), lambda i: (i, 0)),
            pl.BlockSpec((tpn, c), lambda i: (i, 0)),
            pl.BlockSpec((1, 1, c), lambda i: (i, 0, 0)),
            pl.BlockSpec((2 * c, cm), lambda i: (0, 0)),
            pl.BlockSpec((1, cm), lambda i: (0, 0)),
            pl.BlockSpec((cm, c), lambda i: (0, 0)),
            pl.BlockSpec((1, c), lambda i: (0, 0)),
        ],
        out_specs=pl.BlockSpec((1, 1, c), lambda i: (i, 0, 0)),
        out_shape=jax.ShapeDtypeStruct((n, 1, c), _F32),
        compiler_params=pltpu.CompilerParams(
            dimension_semantics=("parallel",)),
    )(psr, psi, s_scale, w1t, b1, w2t, b2)


def _final_body(x_ref, s_ref, g_ref, out_ref):
    c = s_ref.shape[2]
    scale = 1.0 + g_ref[0, 0] * s_ref[0]
    ri = lax.broadcasted_iota(jnp.int32, (c, c), 0)
    ci = lax.broadcasted_iota(jnp.int32, (c, c), 1)
    diag = jnp.where(ri == ci, jnp.broadcast_to(scale, (c, c)), 0.0)
    y = jnp.dot(diag, x_ref[0], preferred_element_type=_F32,
                precision=lax.Precision.HIGHEST)
    out_ref[0] = jnp.maximum(y, 0.0)


def _final_scale(xf, s_scale, gamma):
    n, c, hw = xf.shape
    return pl.pallas_call(
        _final_body,
        grid=(n,),
        in_specs=[
            pl.BlockSpec((1, c, hw), lambda i: (i, 0, 0)),
            pl.BlockSpec((1, 1, c), lambda i: (i, 0, 0)),
            pl.BlockSpec((1, 1), lambda i: (0, 0)),
        ],
        out_specs=pl.BlockSpec((1, c, hw), lambda i: (i, 0, 0)),
        out_shape=jax.ShapeDtypeStruct((n, c, hw), _F32),
        compiler_params=pltpu.CompilerParams(
            dimension_semantics=("parallel",)),
    )(xf, s_scale, gamma)


def kernel(cnn_encoder_output, rgb, ir, gnn_iterations, k,
           rgb_g_W, rgb_g_b, ir_g_W, ir_g_b,
           se_W1, se_b1, se_W2, se_b2, gamma):
    x = cnn_encoder_output
    n, c, h_dim, w_dim = x.shape
    hw = h_dim * w_dim
    xf = x.reshape(n, c, hw)

    rgb_t = rgb.reshape(n, rgb.shape[1], hw)
    ir_t = ir.reshape(n, ir.shape[1], hw)
    ir_pad = jnp.concatenate(
        [ir_t, jnp.zeros((n, rgb.shape[1] - ir.shape[1], hw), _F32)], axis=1)
    mats = jnp.concatenate([rgb_t, ir_pad], axis=0)
    idx_all = _knn_topk(mats)
    idx_rgb, idx_ir = idx_all[:n], idx_all[n:]

    offs = (jnp.arange(n, dtype=jnp.int32) * hw)[:, None, None]
    gidx_rgb = (idx_rgb + offs).reshape(n * hw * _K)
    gidx_ir = (idx_ir + offs).reshape(n * hw * _K)

    h0 = xf.transpose(0, 2, 1)
    wt_rgb = rgb_g_W.T
    wt_ir = ir_g_W.T
    zc = jnp.zeros((1, c), _F32)
    wp = jnp.concatenate([wt_rgb[:c] + wt_rgb[c:], wt_ir[c:]], axis=1)
    bp = jnp.concatenate([rgb_g_b.reshape(1, c), zc], axis=1)
    wq = jnp.concatenate([wt_ir[:c] + wt_ir[c:], wt_rgb[c:]], axis=1)
    bq = jnp.concatenate([ir_g_b.reshape(1, c), zc], axis=1)
    w1t = se_W1.T
    b1 = se_b1.reshape(1, -1)
    w2t = se_W2.T
    b2 = se_b2.reshape(1, c)

    def body(_, s_scale):
        p, q = _feat_tables(h0, s_scale, wp, bp, wq, bq)
        psr, psi = _sc_gather_max(
            p.reshape(n * hw, 2 * c), q.reshape(n * hw, 2 * c),
            gidx_rgb, gidx_ir)
        return _se_update(psr, psi, hw, s_scale, w1t, b1, w2t, b2)

    s_scale = lax.fori_loop(0, gnn_iterations, body,
                            jnp.ones((n, 1, c), _F32))

    out = _final_scale(xf, s_scale, gamma.reshape(1, 1).astype(_F32))
    return out.reshape(n, c, h_dim, w_dim)
